# Initial kernel scaffold; baseline (speedup 1.0000x reference)
import jax
import jax.numpy as jnp
from jax.experimental import pallas as pl


def kernel(z_e, codebook):
    raise NotImplementedError("write your pallas kernel here")


# speedup vs baseline: 1.4914x; 1.4914x over previous
import functools

import jax
import jax.numpy as jnp
from jax import lax
from jax.experimental import pallas as pl
from jax.experimental.pallas import tpu as pltpu
from jax.experimental.pallas import tpu_sc as plsc

NUM_CODES = 512
CODE_DIM = 32
BATCH = 131072


ROW_BLOCK = 1024
GRID = BATCH // ROW_BLOCK


def _argmin_body(z_ref, cbt_ref, idx_ref):
    z = z_ref[...]
    cbt = cbt_ref[...]
    zn = jnp.sum(z * z, axis=1, keepdims=True)
    cbn = jnp.sum(cbt * cbt, axis=0, keepdims=True)
    mm = lax.dot_general(z, cbt, (((1,), (0,)), ((), ())),
                         preferred_element_type=jnp.float32)
    dist = zn - 2.0 * mm + cbn
    m = jnp.min(dist, axis=1, keepdims=True)
    cols = lax.broadcasted_iota(jnp.int32, dist.shape, 1)
    idx = jnp.min(jnp.where(dist == m, cols, NUM_CODES), axis=1)
    idx_ref[0, 0, :] = idx


_argmin_call = pl.pallas_call(
    _argmin_body,
    grid=(GRID,),
    in_specs=[
        pl.BlockSpec((ROW_BLOCK, CODE_DIM), lambda i: (i, 0)),
        pl.BlockSpec((CODE_DIM, NUM_CODES), lambda i: (0, 0)),
    ],
    out_specs=pl.BlockSpec((1, 1, ROW_BLOCK), lambda i: (i, 0, 0)),
    out_shape=jax.ShapeDtypeStruct((GRID, 1, ROW_BLOCK), jnp.int32),
)


NC = 2
NS = 16
NW = NC * NS
B_PER_W = BATCH // NW
CHUNK = 1024
NCHUNKS = B_PER_W // CHUNK
GCHUNK = 128
NG = CHUNK // GCHUNK


def _sc_gather_loss(codebook, indices, z_e):
    mesh = plsc.VectorSubcoreMesh(core_axis_name="c", subcore_axis_name="s")

    @functools.partial(
        pl.kernel,
        mesh=mesh,
        out_type=[
            jax.ShapeDtypeStruct((BATCH, CODE_DIM), jnp.float32),
            jax.ShapeDtypeStruct((NW, 16), jnp.float32),
        ],
        scratch_types=[
            pltpu.VMEM((CHUNK,), jnp.int32),
            pltpu.VMEM((CHUNK, CODE_DIM), jnp.float32),
            pltpu.VMEM((CHUNK, CODE_DIM), jnp.float32),
            pltpu.VMEM((16,), jnp.float32),
            pltpu.SemaphoreType.DMA,
        ],
        compiler_params=pltpu.CompilerParams(use_tc_tiling_on_sc=False),
    )
    def body(cb_hbm, idx_hbm, ze_hbm, zq_hbm, loss_hbm,
             idx_v, rows_v, ze_v, acc_v, sem):
        wid = lax.axis_index("s") * NC + lax.axis_index("c")
        base = wid * B_PER_W
        acc = jnp.zeros((16,), jnp.float32)
        for c in range(NCHUNKS):
            cbase = base + c * CHUNK
            pltpu.sync_copy(idx_hbm.at[pl.ds(cbase, CHUNK)], idx_v)
            copies = [
                pltpu.async_copy(
                    cb_hbm.at[idx_v.at[pl.ds(k * GCHUNK, GCHUNK)]],
                    rows_v.at[pl.ds(k * GCHUNK, GCHUNK)],
                    sem,
                )
                for k in range(NG)
            ]
            pltpu.sync_copy(ze_hbm.at[pl.ds(cbase, CHUNK)], ze_v)
            for cp in copies:
                cp.wait()

            def row_body(r, a):
                d0 = ze_v[r, pl.ds(0, 16)] - rows_v[r, pl.ds(0, 16)]
                d1 = ze_v[r, pl.ds(16, 16)] - rows_v[r, pl.ds(16, 16)]
                return a + d0 * d0 + d1 * d1

            acc = lax.fori_loop(0, CHUNK, row_body, acc)
            pltpu.sync_copy(rows_v, zq_hbm.at[pl.ds(cbase, CHUNK)])
        acc_v[...] = acc
        pltpu.sync_copy(acc_v, loss_hbm.at[wid])

    return body(codebook, indices, z_e)


def kernel(z_e, codebook):
    cbt = codebook.T
    idx3 = _argmin_call(z_e, cbt)
    indices = idx3.reshape(BATCH)
    z_q, partials = _sc_gather_loss(codebook, indices, z_e)
    loss = jnp.sum(partials) / jnp.float32(BATCH * CODE_DIM)
    return (z_q, indices, loss)


# speedup vs baseline: 2.3139x; 1.5514x over previous
import functools

import jax
import jax.numpy as jnp
from jax import lax
from jax.experimental import pallas as pl
from jax.experimental.pallas import tpu as pltpu
from jax.experimental.pallas import tpu_sc as plsc

NUM_CODES = 512
CODE_DIM = 32
BATCH = 131072


ROW_BLOCK = 2048
GRID = BATCH // ROW_BLOCK


def _argmin_body(zt_ref, cb2_ref, cbtb_ref, idx_ref, lp_ref):
    zt = zt_ref[...]
    cb2 = cb2_ref[...]
    cbtb = cbtb_ref[...]
    znt = jnp.sum(zt * zt, axis=0, keepdims=True)
    cbnt = 0.25 * jnp.sum(cb2 * cb2, axis=1, keepdims=True)
    mm2t = lax.dot_general(cb2, zt, (((1,), (0,)), ((), ())),
                           preferred_element_type=jnp.float32)
    dist = (znt + mm2t) + cbnt
    m = jnp.min(dist, axis=0, keepdims=True)
    onehot = jnp.where(dist == m, 1.0, 0.0).astype(jnp.bfloat16)
    col = lax.broadcasted_iota(jnp.int32, (2, NUM_CODES), 1)
    row = lax.broadcasted_iota(jnp.int32, (2, NUM_CODES), 0)
    hl = jnp.where(row == 0, col >> 8, col & 255).astype(jnp.bfloat16)
    idx2 = lax.dot_general(hl, onehot, (((1,), (0,)), ((), ())),
                           preferred_element_type=jnp.float32)
    idxf = idx2[0:1, :] * 256.0 + idx2[1:2, :]
    idx = jnp.minimum(idxf, float(NUM_CODES - 1)).astype(jnp.int32)
    idx_ref[0, 0, :] = idx[0, :]
    zqt = lax.dot_general(cbtb, onehot, (((1,), (0,)), ((), ())),
                          preferred_element_type=jnp.float32)
    diff = zt - zqt
    lp_ref[0, 0, :] = jnp.broadcast_to(jnp.sum(diff * diff), (128,))


_argmin_call = pl.pallas_call(
    _argmin_body,
    grid=(GRID,),
    in_specs=[
        pl.BlockSpec((CODE_DIM, ROW_BLOCK), lambda i: (0, i)),
        pl.BlockSpec((NUM_CODES, CODE_DIM), lambda i: (0, 0)),
        pl.BlockSpec((CODE_DIM, NUM_CODES), lambda i: (0, 0)),
    ],
    out_specs=[
        pl.BlockSpec((1, 1, ROW_BLOCK), lambda i: (i, 0, 0)),
        pl.BlockSpec((1, 1, 128), lambda i: (i, 0, 0)),
    ],
    out_shape=[
        jax.ShapeDtypeStruct((GRID, 1, ROW_BLOCK), jnp.int32),
        jax.ShapeDtypeStruct((GRID, 1, 128), jnp.float32),
    ],
)


NC = 2
NS = 16
NW = NC * NS
B_PER_W = BATCH // NW
CHUNK = 1024
NCHUNKS = B_PER_W // CHUNK
GCHUNK = 128
NG = CHUNK // GCHUNK


def _sc_gather(codebook, indices):
    mesh = plsc.VectorSubcoreMesh(core_axis_name="c", subcore_axis_name="s")

    @functools.partial(
        pl.kernel,
        mesh=mesh,
        out_type=jax.ShapeDtypeStruct((BATCH, CODE_DIM), jnp.float32),
        scratch_types=[
            pltpu.VMEM((CHUNK,), jnp.int32),
            pltpu.VMEM((CHUNK,), jnp.int32),
            pltpu.VMEM((CHUNK, CODE_DIM), jnp.float32),
            pltpu.VMEM((CHUNK, CODE_DIM), jnp.float32),
            pltpu.SemaphoreType.DMA,
            pltpu.SemaphoreType.DMA,
            pltpu.SemaphoreType.DMA,
        ],
        compiler_params=pltpu.CompilerParams(use_tc_tiling_on_sc=False),
    )
    def body(cb_hbm, idx_hbm, zq_hbm,
             idx_v0, idx_v1, rows_v0, rows_v1, sem_g, sem_o0, sem_o1):
        wid = lax.axis_index("s") * NC + lax.axis_index("c")
        base = wid * B_PER_W
        bufs = [(idx_v0, rows_v0, sem_o0), (idx_v1, rows_v1, sem_o1)]
        out_copies = [None, None]
        for c in range(NCHUNKS):
            idx_v, rows_v, sem_o = bufs[c % 2]
            if out_copies[c % 2] is not None:
                out_copies[c % 2].wait()
            cbase = base + c * CHUNK
            pltpu.sync_copy(idx_hbm.at[pl.ds(cbase, CHUNK)], idx_v)
            gathers = [
                pltpu.async_copy(
                    cb_hbm.at[idx_v.at[pl.ds(k * GCHUNK, GCHUNK)]],
                    rows_v.at[pl.ds(k * GCHUNK, GCHUNK)],
                    sem_g,
                )
                for k in range(NG)
            ]
            for g in gathers:
                g.wait()
            out_copies[c % 2] = pltpu.async_copy(
                rows_v, zq_hbm.at[pl.ds(cbase, CHUNK)], sem_o)
        for cp in out_copies:
            if cp is not None:
                cp.wait()

    return body(codebook, indices)


def kernel(z_e, codebook):
    zt = z_e.T
    cb2 = codebook * jnp.float32(-2.0)
    cbtb = codebook.T.astype(jnp.bfloat16)
    idx3, lp = _argmin_call(zt, cb2, cbtb)
    indices = idx3.reshape(BATCH)
    z_q = _sc_gather(codebook, indices)
    loss = jnp.sum(lp[:, 0, 0]) / jnp.float32(BATCH * CODE_DIM)
    return (z_q, indices, loss)


# speedup vs baseline: 2.7710x; 1.1976x over previous
import functools

import jax
import jax.numpy as jnp
from jax import lax
from jax.experimental import pallas as pl
from jax.experimental.pallas import tpu as pltpu
from jax.experimental.pallas import tpu_sc as plsc

NUM_CODES = 512
CODE_DIM = 32
BATCH = 131072


ROW_BLOCK = 2048
GRID = BATCH // ROW_BLOCK


def _argmin_body(z_ref, cb2_ref, idx_ref, lp_ref):
    z = z_ref[...]
    cb2 = cb2_ref[...]
    ones = jnp.ones((1, CODE_DIM), jnp.float32)
    znt = lax.dot_general(ones, z * z, (((1,), (1,)), ((), ())),
                          preferred_element_type=jnp.float32)
    cbnt = 0.25 * jnp.sum(cb2 * cb2, axis=1, keepdims=True)
    mm2t = lax.dot_general(cb2, z, (((1,), (1,)), ((), ())),
                           preferred_element_type=jnp.float32)
    dist = (znt + mm2t) + cbnt
    m = jnp.min(dist, axis=0, keepdims=True)
    onehot = jnp.where(dist == m, 1.0, 0.0).astype(jnp.bfloat16)
    col = lax.broadcasted_iota(jnp.int32, (2, NUM_CODES), 1)
    row = lax.broadcasted_iota(jnp.int32, (2, NUM_CODES), 0)
    hl = jnp.where(row == 0, col >> 8, col & 255).astype(jnp.bfloat16)
    idx2 = lax.dot_general(hl, onehot, (((1,), (0,)), ((), ())),
                           preferred_element_type=jnp.float32)
    idxf = idx2[0:1, :] * 256.0 + idx2[1:2, :]
    idx = jnp.minimum(idxf, float(NUM_CODES - 1)).astype(jnp.int32)
    idx_ref[0, 0, :] = idx[0, :]
    lp_ref[0, 0, :] = jnp.broadcast_to(jnp.sum(m), (128,))


_argmin_call = pl.pallas_call(
    _argmin_body,
    grid=(GRID,),
    in_specs=[
        pl.BlockSpec((ROW_BLOCK, CODE_DIM), lambda i: (i, 0)),
        pl.BlockSpec((NUM_CODES, CODE_DIM), lambda i: (0, 0)),
    ],
    out_specs=[
        pl.BlockSpec((1, 1, ROW_BLOCK), lambda i: (i, 0, 0)),
        pl.BlockSpec((1, 1, 128), lambda i: (i, 0, 0)),
    ],
    out_shape=[
        jax.ShapeDtypeStruct((GRID, 1, ROW_BLOCK), jnp.int32),
        jax.ShapeDtypeStruct((GRID, 1, 128), jnp.float32),
    ],
)


NC = 2
NS = 16
NW = NC * NS
B_PER_W = BATCH // NW
CHUNK = 1024
NCHUNKS = B_PER_W // CHUNK
GCHUNK = 128
NG = CHUNK // GCHUNK


def _sc_gather(codebook, indices):
    mesh = plsc.VectorSubcoreMesh(core_axis_name="c", subcore_axis_name="s")

    @functools.partial(
        pl.kernel,
        mesh=mesh,
        out_type=jax.ShapeDtypeStruct((BATCH, CODE_DIM), jnp.float32),
        scratch_types=[
            pltpu.VMEM_SHARED((NUM_CODES, CODE_DIM), jnp.float32),
            pltpu.VMEM((CHUNK,), jnp.int32),
            pltpu.VMEM((CHUNK,), jnp.int32),
            pltpu.VMEM((CHUNK, CODE_DIM), jnp.float32),
            pltpu.VMEM((CHUNK, CODE_DIM), jnp.float32),
            pltpu.SemaphoreType.DMA,
            pltpu.SemaphoreType.DMA,
            pltpu.SemaphoreType.DMA,
        ],
        compiler_params=pltpu.CompilerParams(use_tc_tiling_on_sc=False),
    )
    def body(cb_hbm, idx_hbm, zq_hbm,
             cb_v, idx_v0, idx_v1, rows_v0, rows_v1, sem_g, sem_o0, sem_o1):
        sid = lax.axis_index("s")
        wid = sid * NC + lax.axis_index("c")
        base = wid * B_PER_W

        @pl.when(sid == 0)
        def _():
            pltpu.sync_copy(cb_hbm, cb_v)

        plsc.subcore_barrier()
        bufs = [(idx_v0, rows_v0, sem_o0), (idx_v1, rows_v1, sem_o1)]
        out_copies = [None, None]
        for c in range(NCHUNKS):
            idx_v, rows_v, sem_o = bufs[c % 2]
            if out_copies[c % 2] is not None:
                out_copies[c % 2].wait()
            cbase = base + c * CHUNK
            pltpu.sync_copy(idx_hbm.at[pl.ds(cbase, CHUNK)], idx_v)
            gathers = [
                pltpu.async_copy(
                    cb_v.at[idx_v.at[pl.ds(k * GCHUNK, GCHUNK)]],
                    rows_v.at[pl.ds(k * GCHUNK, GCHUNK)],
                    sem_g,
                )
                for k in range(NG)
            ]
            for g in gathers:
                g.wait()
            out_copies[c % 2] = pltpu.async_copy(
                rows_v, zq_hbm.at[pl.ds(cbase, CHUNK)], sem_o)
        for cp in out_copies:
            if cp is not None:
                cp.wait()

    return body(codebook, indices)


def kernel(z_e, codebook):
    cb2 = codebook * jnp.float32(-2.0)
    idx3, lp = _argmin_call(z_e, cb2)
    indices = idx3.reshape(BATCH)
    z_q = _sc_gather(codebook, indices)
    loss = jnp.sum(lp[:, 0, 0]) / jnp.float32(BATCH * CODE_DIM)
    return (z_q, indices, loss)


# speedup vs baseline: 3.5403x; 1.2776x over previous
import functools

import jax
import jax.numpy as jnp
from jax import lax
from jax.experimental import pallas as pl
from jax.experimental.pallas import tpu as pltpu
from jax.experimental.pallas import tpu_sc as plsc

NUM_CODES = 512
CODE_DIM = 32
BATCH = 131072


ROW_BLOCK = 2048
GRID = BATCH // ROW_BLOCK


def _argmin_body(zt_ref, cb2_ref, idx_ref, lp_ref):
    zt = zt_ref[...]
    cb2 = cb2_ref[...]
    znt = jnp.sum(zt * zt, axis=0, keepdims=True)
    cbnt = 0.25 * jnp.sum(cb2 * cb2, axis=1, keepdims=True)
    mm2t = lax.dot_general(cb2, zt, (((1,), (0,)), ((), ())),
                           preferred_element_type=jnp.float32)
    dist = (znt + mm2t) + cbnt
    m = jnp.min(dist, axis=0, keepdims=True)
    onehot = jnp.where(dist == m, 1.0, 0.0).astype(jnp.bfloat16)
    col = lax.broadcasted_iota(jnp.int32, (2, NUM_CODES), 1)
    row = lax.broadcasted_iota(jnp.int32, (2, NUM_CODES), 0)
    hl = jnp.where(row == 0, col >> 8, col & 255).astype(jnp.bfloat16)
    idx2 = lax.dot_general(hl, onehot, (((1,), (0,)), ((), ())),
                           preferred_element_type=jnp.float32)
    idxf = idx2[0:1, :] * 256.0 + idx2[1:2, :]
    idx = jnp.minimum(idxf, float(NUM_CODES - 1)).astype(jnp.int32)
    idx_ref[0, 0, :] = idx[0, :]
    lp_ref[0, 0, :] = jnp.broadcast_to(jnp.sum(m), (128,))


_argmin_call = pl.pallas_call(
    _argmin_body,
    grid=(GRID,),
    in_specs=[
        pl.BlockSpec((CODE_DIM, ROW_BLOCK), lambda i: (0, i)),
        pl.BlockSpec((NUM_CODES, CODE_DIM), lambda i: (0, 0)),
    ],
    out_specs=[
        pl.BlockSpec((1, 1, ROW_BLOCK), lambda i: (i, 0, 0)),
        pl.BlockSpec((1, 1, 128), lambda i: (i, 0, 0)),
    ],
    out_shape=[
        jax.ShapeDtypeStruct((GRID, 1, ROW_BLOCK), jnp.int32),
        jax.ShapeDtypeStruct((GRID, 1, 128), jnp.float32),
    ],
)


NC = 2
NS = 16
NW = NC * NS
B_PER_W = BATCH // NW
CHUNK = 1024
NCHUNKS = B_PER_W // CHUNK
GCHUNK = 128
NG = CHUNK // GCHUNK


def _sc_gather(codebook, indices):
    mesh = plsc.VectorSubcoreMesh(core_axis_name="c", subcore_axis_name="s")

    @functools.partial(
        pl.kernel,
        mesh=mesh,
        out_type=jax.ShapeDtypeStruct((BATCH, CODE_DIM), jnp.float32),
        scratch_types=[
            pltpu.VMEM_SHARED((NUM_CODES, CODE_DIM), jnp.float32),
            pltpu.VMEM((CHUNK,), jnp.int32),
            pltpu.VMEM((CHUNK,), jnp.int32),
            pltpu.VMEM((CHUNK, CODE_DIM), jnp.float32),
            pltpu.VMEM((CHUNK, CODE_DIM), jnp.float32),
            pltpu.SemaphoreType.DMA,
            pltpu.SemaphoreType.DMA,
            pltpu.SemaphoreType.DMA,
        ],
        compiler_params=pltpu.CompilerParams(use_tc_tiling_on_sc=False),
    )
    def body(cb_hbm, idx_hbm, zq_hbm,
             cb_v, idx_v0, idx_v1, rows_v0, rows_v1, sem_g, sem_o0, sem_o1):
        sid = lax.axis_index("s")
        wid = sid * NC + lax.axis_index("c")
        base = wid * B_PER_W

        @pl.when(sid == 0)
        def _():
            pltpu.sync_copy(cb_hbm, cb_v)

        plsc.subcore_barrier()
        bufs = [(idx_v0, rows_v0, sem_o0), (idx_v1, rows_v1, sem_o1)]
        out_copies = [None, None]
        for c in range(NCHUNKS):
            idx_v, rows_v, sem_o = bufs[c % 2]
            if out_copies[c % 2] is not None:
                out_copies[c % 2].wait()
            cbase = base + c * CHUNK
            pltpu.sync_copy(idx_hbm.at[pl.ds(cbase, CHUNK)], idx_v)
            gathers = [
                pltpu.async_copy(
                    cb_v.at[idx_v.at[pl.ds(k * GCHUNK, GCHUNK)]],
                    rows_v.at[pl.ds(k * GCHUNK, GCHUNK)],
                    sem_g,
                )
                for k in range(NG)
            ]
            for g in gathers:
                g.wait()
            out_copies[c % 2] = pltpu.async_copy(
                rows_v, zq_hbm.at[pl.ds(cbase, CHUNK)], sem_o)
        for cp in out_copies:
            if cp is not None:
                cp.wait()

    return body(codebook, indices)


def kernel(z_e, codebook):
    zt = z_e.T
    cb2 = codebook * jnp.float32(-2.0)
    idx3, lp = _argmin_call(zt, cb2)
    indices = idx3.reshape(BATCH)
    z_q = _sc_gather(codebook, indices)
    loss = jnp.sum(lp[:, 0, 0]) / jnp.float32(BATCH * CODE_DIM)
    return (z_q, indices, loss)


# speedup vs baseline: 3.6761x; 1.0384x over previous
import functools

import jax
import jax.numpy as jnp
from jax import lax
from jax.experimental import pallas as pl
from jax.experimental.pallas import tpu as pltpu
from jax.experimental.pallas import tpu_sc as plsc

NUM_CODES = 512
CODE_DIM = 32
BATCH = 131072


ROW_BLOCK = 4096
GRID = BATCH // ROW_BLOCK


def _argmin_body(zt_ref, cb2_ref, idx_ref, lp_ref):
    zt = zt_ref[...]
    cb2 = cb2_ref[...]
    znt = jnp.sum(zt * zt, axis=0, keepdims=True)
    cbnt = 0.25 * jnp.sum(cb2 * cb2, axis=1, keepdims=True)
    mm2t = lax.dot_general(cb2, zt, (((1,), (0,)), ((), ())),
                           preferred_element_type=jnp.float32)
    dist = (znt + mm2t) + cbnt
    m = jnp.min(dist, axis=0, keepdims=True)
    onehot = jnp.where(dist == m, 1.0, 0.0)
    col = lax.broadcasted_iota(jnp.int32, (2, NUM_CODES), 1)
    row = lax.broadcasted_iota(jnp.int32, (2, NUM_CODES), 0)
    hl = jnp.where(row == 0, col >> 8, col & 255).astype(jnp.float32)
    idx2 = lax.dot_general(hl, onehot, (((1,), (0,)), ((), ())),
                           preferred_element_type=jnp.float32)
    idxf = idx2[0:1, :] * 256.0 + idx2[1:2, :]
    idx = jnp.minimum(idxf, float(NUM_CODES - 1)).astype(jnp.int32)
    idx_ref[0, 0, :] = idx[0, :]
    lp_ref[0, 0, :] = jnp.broadcast_to(jnp.sum(m), (128,))


_argmin_call = pl.pallas_call(
    _argmin_body,
    grid=(GRID,),
    in_specs=[
        pl.BlockSpec((CODE_DIM, ROW_BLOCK), lambda i: (0, i)),
        pl.BlockSpec((NUM_CODES, CODE_DIM), lambda i: (0, 0)),
    ],
    out_specs=[
        pl.BlockSpec((1, 1, ROW_BLOCK), lambda i: (i, 0, 0)),
        pl.BlockSpec((1, 1, 128), lambda i: (i, 0, 0)),
    ],
    out_shape=[
        jax.ShapeDtypeStruct((GRID, 1, ROW_BLOCK), jnp.int32),
        jax.ShapeDtypeStruct((GRID, 1, 128), jnp.float32),
    ],
)


NC = 2
NS = 16
NW = NC * NS
B_PER_W = BATCH // NW
CHUNK = 1024
NCHUNKS = B_PER_W // CHUNK
GCHUNK = 128
NG = CHUNK // GCHUNK


def _sc_gather(codebook, indices):
    mesh = plsc.VectorSubcoreMesh(core_axis_name="c", subcore_axis_name="s")

    @functools.partial(
        pl.kernel,
        mesh=mesh,
        out_type=jax.ShapeDtypeStruct((BATCH, CODE_DIM), jnp.float32),
        scratch_types=[
            pltpu.VMEM_SHARED((NUM_CODES, CODE_DIM), jnp.float32),
            pltpu.VMEM((CHUNK,), jnp.int32),
            pltpu.VMEM((CHUNK,), jnp.int32),
            pltpu.VMEM((CHUNK, CODE_DIM), jnp.float32),
            pltpu.VMEM((CHUNK, CODE_DIM), jnp.float32),
            pltpu.SemaphoreType.DMA,
            pltpu.SemaphoreType.DMA,
            pltpu.SemaphoreType.DMA,
        ],
        compiler_params=pltpu.CompilerParams(use_tc_tiling_on_sc=False),
    )
    def body(cb_hbm, idx_hbm, zq_hbm,
             cb_v, idx_v0, idx_v1, rows_v0, rows_v1, sem_g, sem_o0, sem_o1):
        sid = lax.axis_index("s")
        wid = sid * NC + lax.axis_index("c")
        base = wid * B_PER_W

        @pl.when(sid == 0)
        def _():
            pltpu.sync_copy(cb_hbm, cb_v)

        plsc.subcore_barrier()
        bufs = [(idx_v0, rows_v0, sem_o0), (idx_v1, rows_v1, sem_o1)]
        out_copies = [None, None]
        for c in range(NCHUNKS):
            idx_v, rows_v, sem_o = bufs[c % 2]
            if out_copies[c % 2] is not None:
                out_copies[c % 2].wait()
            cbase = base + c * CHUNK
            pltpu.sync_copy(idx_hbm.at[pl.ds(cbase, CHUNK)], idx_v)
            gathers = [
                pltpu.async_copy(
                    cb_v.at[idx_v.at[pl.ds(k * GCHUNK, GCHUNK)]],
                    rows_v.at[pl.ds(k * GCHUNK, GCHUNK)],
                    sem_g,
                )
                for k in range(NG)
            ]
            for g in gathers:
                g.wait()
            out_copies[c % 2] = pltpu.async_copy(
                rows_v, zq_hbm.at[pl.ds(cbase, CHUNK)], sem_o)
        for cp in out_copies:
            if cp is not None:
                cp.wait()

    return body(codebook, indices)


def kernel(z_e, codebook):
    zt = z_e.T
    cb2 = codebook * jnp.float32(-2.0)
    idx3, lp = _argmin_call(zt, cb2)
    indices = idx3.reshape(BATCH)
    z_q = _sc_gather(codebook, indices)
    loss = jnp.sum(lp[:, 0, 0]) / jnp.float32(BATCH * CODE_DIM)
    return (z_q, indices, loss)


# speedup vs baseline: 3.7703x; 1.0256x over previous
import functools

import jax
import jax.numpy as jnp
from jax import lax
from jax.experimental import pallas as pl
from jax.experimental.pallas import tpu as pltpu
from jax.experimental.pallas import tpu_sc as plsc

NUM_CODES = 512
CODE_DIM = 32
BATCH = 131072


ROW_BLOCK = 4096
GRID = BATCH // ROW_BLOCK


def _argmin_body(zt_ref, cb2_ref, idx_ref, lp_ref):
    zt = zt_ref[...]
    cb2 = cb2_ref[...]
    znt = jnp.sum(zt * zt, axis=0, keepdims=True)
    cbnt = 0.25 * jnp.sum(cb2 * cb2, axis=1, keepdims=True)
    mm2t = lax.dot_general(cb2, zt, (((1,), (0,)), ((), ())),
                           preferred_element_type=jnp.float32)
    dist = (znt + mm2t) + cbnt
    m = jnp.min(dist, axis=0, keepdims=True)
    onehot = jnp.where(dist == m, 1.0, 0.0)
    col = lax.broadcasted_iota(jnp.int32, (2, NUM_CODES), 1)
    row = lax.broadcasted_iota(jnp.int32, (2, NUM_CODES), 0)
    hl = jnp.where(row == 0, col >> 8, col & 255).astype(jnp.float32)
    idx2 = lax.dot_general(hl, onehot, (((1,), (0,)), ((), ())),
                           preferred_element_type=jnp.float32)
    idxf = idx2[0:1, :] * 256.0 + idx2[1:2, :]
    idx = jnp.minimum(idxf, float(NUM_CODES - 1)).astype(jnp.int32)
    idx_ref[0, 0, :] = idx[0, :]
    lp_ref[0, 0, :] = jnp.broadcast_to(jnp.sum(m), (128,))


NSPLIT = 2
HBATCH = BATCH // NSPLIT
HGRID = HBATCH // ROW_BLOCK


def _make_argmin_call(off):
    return pl.pallas_call(
        _argmin_body,
        grid=(HGRID,),
        in_specs=[
            pl.BlockSpec((CODE_DIM, ROW_BLOCK), lambda i: (0, i + off)),
            pl.BlockSpec((NUM_CODES, CODE_DIM), lambda i: (0, 0)),
        ],
        out_specs=[
            pl.BlockSpec((1, 1, ROW_BLOCK), lambda i: (i, 0, 0)),
            pl.BlockSpec((1, 1, 128), lambda i: (i, 0, 0)),
        ],
        out_shape=[
            jax.ShapeDtypeStruct((HGRID, 1, ROW_BLOCK), jnp.int32),
            jax.ShapeDtypeStruct((HGRID, 1, 128), jnp.float32),
        ],
    )


_argmin_calls = [_make_argmin_call(h * HGRID) for h in range(NSPLIT)]


NC = 2
NS = 16
NW = NC * NS
B_PER_W = HBATCH // NW
CHUNK = 1024
NCHUNKS = B_PER_W // CHUNK
GCHUNK = 128
NG = CHUNK // GCHUNK


def _sc_gather(codebook, indices):
    mesh = plsc.VectorSubcoreMesh(core_axis_name="c", subcore_axis_name="s")

    @functools.partial(
        pl.kernel,
        mesh=mesh,
        out_type=jax.ShapeDtypeStruct((HBATCH, CODE_DIM), jnp.float32),
        scratch_types=[
            pltpu.VMEM_SHARED((NUM_CODES, CODE_DIM), jnp.float32),
            pltpu.VMEM((CHUNK,), jnp.int32),
            pltpu.VMEM((CHUNK,), jnp.int32),
            pltpu.VMEM((CHUNK, CODE_DIM), jnp.float32),
            pltpu.VMEM((CHUNK, CODE_DIM), jnp.float32),
            pltpu.SemaphoreType.DMA,
            pltpu.SemaphoreType.DMA,
            pltpu.SemaphoreType.DMA,
        ],
        compiler_params=pltpu.CompilerParams(use_tc_tiling_on_sc=False),
    )
    def body(cb_hbm, idx_hbm, zq_hbm,
             cb_v, idx_v0, idx_v1, rows_v0, rows_v1, sem_g, sem_o0, sem_o1):
        sid = lax.axis_index("s")
        wid = sid * NC + lax.axis_index("c")
        base = wid * B_PER_W

        @pl.when(sid == 0)
        def _():
            pltpu.sync_copy(cb_hbm, cb_v)

        plsc.subcore_barrier()
        bufs = [(idx_v0, rows_v0, sem_o0), (idx_v1, rows_v1, sem_o1)]
        out_copies = [None, None]
        for c in range(NCHUNKS):
            idx_v, rows_v, sem_o = bufs[c % 2]
            if out_copies[c % 2] is not None:
                out_copies[c % 2].wait()
            cbase = base + c * CHUNK
            pltpu.sync_copy(idx_hbm.at[pl.ds(cbase, CHUNK)], idx_v)
            gathers = [
                pltpu.async_copy(
                    cb_v.at[idx_v.at[pl.ds(k * GCHUNK, GCHUNK)]],
                    rows_v.at[pl.ds(k * GCHUNK, GCHUNK)],
                    sem_g,
                )
                for k in range(NG)
            ]
            for g in gathers:
                g.wait()
            out_copies[c % 2] = pltpu.async_copy(
                rows_v, zq_hbm.at[pl.ds(cbase, CHUNK)], sem_o)
        for cp in out_copies:
            if cp is not None:
                cp.wait()

    return body(codebook, indices)


def kernel(z_e, codebook):
    zt = z_e.T
    cb2 = codebook * jnp.float32(-2.0)
    idx_halves, zq_halves, lp_halves = [], [], []
    for h in range(NSPLIT):
        idx3, lp = _argmin_calls[h](zt, cb2)
        ind = idx3.reshape(HBATCH)
        idx_halves.append(ind)
        lp_halves.append(lp)
        zq_halves.append(_sc_gather(codebook, ind))
    indices = jnp.concatenate(idx_halves)
    z_q = jnp.concatenate(zq_halves, axis=0)
    lp_sum = sum(jnp.sum(lp[:, 0, 0]) for lp in lp_halves)
    loss = lp_sum / jnp.float32(BATCH * CODE_DIM)
    return (z_q, indices, loss)
